# baseline (device time: 84760 ns/iter reference)
import jax
import jax.numpy as jnp
from jax import lax
from jax.experimental import pallas as pl
from jax.experimental.pallas import tpu as pltpu

N_DEV = 16
NZ = 4
NF = 4
NQ = 4


def kernel(x, w_mat):
    m_per, k = x.shape
    _, n_per = w_mat.shape
    m_glob = N_DEV * m_per
    m_q = m_per // NQ

    def body(x_ref, w_ref, out_ref, gather_ref, copy_sem,
             su, ru, sd, rd,
             sA, rA, sB, rB,
             sC, rC, sD, rD,
             fsu, fru, fsd, frd):
        my = lax.axis_index("i")
        jj = lax.rem(my, NF)
        zz = my // NF
        j_left = lax.rem(jj + (NF - 1), NF)
        j_right = lax.rem(jj + 1, NF)
        p_right = NF * zz + j_right
        p_left = NF * zz + j_left
        p_up = my + NF
        p_dn = my - NF

        has_up = zz < NZ - 1
        has_dn = zz > 0

        started = []

        def start(desc, cond=None):
            if cond is None:
                desc.start()
            else:
                pl.when(cond)(lambda: desc.start())
            started.append((desc, cond))

        barrier_sem = pltpu.get_barrier_semaphore()
        for nbr in (p_left, p_right):
            pl.semaphore_signal(barrier_sem, inc=1, device_id=(nbr,),
                                device_id_type=pl.DeviceIdType.MESH)
        pl.when(has_up)(lambda: pl.semaphore_signal(
            barrier_sem, inc=1, device_id=(p_up,),
            device_id_type=pl.DeviceIdType.MESH))
        pl.when(has_dn)(lambda: pl.semaphore_signal(
            barrier_sem, inc=1, device_id=(p_dn,),
            device_id_type=pl.DeviceIdType.MESH))
        pl.semaphore_wait(barrier_sem, 2)
        pl.when(has_up)(lambda: pl.semaphore_wait(barrier_sem, 1))
        pl.when(has_dn)(lambda: pl.semaphore_wait(barrier_sem, 1))

        copies = [
            pltpu.make_async_copy(
                x_ref.at[pl.ds(q * m_q, m_q)],
                gather_ref.at[jj, zz, q],
                copy_sem,
            )
            for q in range(NQ)
        ]
        for c in copies:
            c.start()
        for c in copies:
            c.wait()

        def z_send(z_src, q, up):
            return pltpu.make_async_remote_copy(
                src_ref=gather_ref.at[jj, z_src, q],
                dst_ref=gather_ref.at[jj, z_src, q],
                send_sem=(su if up else sd).at[z_src, q],
                recv_sem=(ru if up else rd).at[z_src, q],
                device_id=(p_up if up else p_dn,),
                device_id_type=pl.DeviceIdType.MESH,
            )

        def face_send(j_src, z_src, q, to_right, sems):
            s_sem, r_sem = sems
            return pltpu.make_async_remote_copy(
                src_ref=gather_ref.at[j_src, z_src, q],
                dst_ref=gather_ref.at[j_src, z_src, q],
                send_sem=s_sem.at[z_src, q],
                recv_sem=r_sem.at[z_src, q],
                device_id=(p_right if to_right else p_left,),
                device_id_type=pl.DeviceIdType.MESH,
            )

        def gemm_rows(piece_rows, origin):
            out_ref[pl.ds(origin * m_per, m_per), :] = jnp.maximum(
                jnp.dot(piece_rows, w_ref[...],
                        preferred_element_type=jnp.float32),
                0.0,
            )

        def gemm_chunk(j_src, z_src):
            gemm_rows(
                gather_ref[j_src, z_src].reshape(m_per, k),
                NF * z_src + j_src,
            )

        for q in range(NQ):
            start(z_send(zz, q, up=True), has_up)
            start(z_send(zz, q, up=False), has_dn)
            start(face_send(jj, zz, q, True, (sA, rA)))
            start(face_send(jj, zz, q, False, (sB, rB)))

        gemm_rows(x_ref[...], my)

        for d in range(1, NZ):
            z_lo = zz - d
            z_hi = zz + d
            lo_ok = z_lo >= 0
            hi_ok = z_hi <= NZ - 1
            for q in range(NQ):
                def lo_work(q=q):
                    pltpu.make_async_remote_copy(
                        src_ref=gather_ref.at[jj, z_lo, q],
                        dst_ref=gather_ref.at[jj, z_lo, q],
                        send_sem=su.at[z_lo, q],
                        recv_sem=ru.at[z_lo, q],
                        device_id=(p_dn,),
                        device_id_type=pl.DeviceIdType.MESH,
                    ).wait_recv()
                pl.when(lo_ok)(lo_work)
                start(z_send(z_lo, q, up=True),
                      jnp.logical_and(lo_ok, has_up))
                start(face_send(jj, z_lo, q, True, (sA, rA)), lo_ok)
                start(face_send(jj, z_lo, q, False, (sB, rB)), lo_ok)

                def hi_work(q=q):
                    pltpu.make_async_remote_copy(
                        src_ref=gather_ref.at[jj, z_hi, q],
                        dst_ref=gather_ref.at[jj, z_hi, q],
                        send_sem=sd.at[z_hi, q],
                        recv_sem=rd.at[z_hi, q],
                        device_id=(p_up,),
                        device_id_type=pl.DeviceIdType.MESH,
                    ).wait_recv()
                pl.when(hi_ok)(hi_work)
                start(z_send(z_hi, q, up=False),
                      jnp.logical_and(hi_ok, has_dn))
                start(face_send(jj, z_hi, q, True, (sA, rA)), hi_ok)
                start(face_send(jj, z_hi, q, False, (sB, rB)), hi_ok)
            pl.when(lo_ok)(lambda z=z_lo: gemm_chunk(jj, z))
            pl.when(hi_ok)(lambda z=z_hi: gemm_chunk(jj, z))

        z_order = [(zz, None)] + [
            (zz - d, zz + d) for d in range(1, NZ)
        ]

        def plane_pieces(process):
            for z_src, z_alt in z_order:
                if z_alt is None:
                    process(z_src, None)
                else:
                    process(z_src, z_src >= 0)
                    process(z_alt, z_alt <= NZ - 1)

        def and_cond(cond, extra):
            return extra if cond is None else jnp.logical_and(cond, extra)

        def neighbor_stacks(z_src, cond):
            for q in range(NQ):
                def workA(q=q):
                    face_send(j_left, z_src, q, True, (sA, rA)).wait_recv()
                if cond is None:
                    workA()
                else:
                    pl.when(cond)(workA)
                if q == 0:
                    start(face_send(j_left, z_src, q, True, (sC, rC)),
                          cond)
                elif q == 1:
                    start(face_send(j_left, z_src, q, True, (sC, rC)),
                          and_cond(cond, z_src == zz))

                def workB(q=q):
                    face_send(j_right, z_src, q, False, (sB, rB)).wait_recv()
                if cond is None:
                    workB()
                else:
                    pl.when(cond)(workB)
                if q == 2:
                    start(face_send(j_right, z_src, q, False, (sD, rD)),
                          cond)
                elif q == 3:
                    start(face_send(j_right, z_src, q, False, (sD, rD)),
                          and_cond(cond, z_src == zz))

            def gemms():
                gemm_chunk(j_left, z_src)
                gemm_chunk(j_right, z_src)
            if cond is None:
                gemms()
            else:
                pl.when(cond)(gemms)

        plane_pieces(neighbor_stacks)

        j_opp = lax.rem(jj + 2, NF)

        def foreign_z(z_src, q, up):
            return pltpu.make_async_remote_copy(
                src_ref=gather_ref.at[j_opp, z_src, q],
                dst_ref=gather_ref.at[j_opp, z_src, q],
                send_sem=(fsu if up else fsd).at[z_src, q],
                recv_sem=(fru if up else frd).at[z_src, q],
                device_id=(p_up if up else p_dn,),
                device_id_type=pl.DeviceIdType.MESH,
            )

        def opposite_quarters(z_src, cond):
            for q in range(NQ):
                entry = and_cond(cond, z_src == zz) if q in (1, 3) else cond

                def work(q=q):
                    if q < NQ // 2:
                        face_send(j_opp, z_src, q, True, (sC, rC)).wait_recv()
                    else:
                        face_send(j_opp, z_src, q, False, (sD, rD)).wait_recv()
                if entry is None:
                    work()
                else:
                    pl.when(entry)(work)
                if q in (1, 3):
                    start(foreign_z(z_src, q, up=True),
                          and_cond(entry, has_up))
                    start(foreign_z(z_src, q, up=False),
                          and_cond(entry, has_dn))

        plane_pieces(opposite_quarters)

        for d in range(1, NZ):
            z_lo = zz - d
            z_hi = zz + d
            lo_ok = z_lo >= 0
            hi_ok = z_hi <= NZ - 1
            for q in (1, 3):
                pl.when(lo_ok)(
                    lambda z=z_lo, q=q: foreign_z(z, q, up=True).wait_recv()
                )
                start(foreign_z(z_lo, q, up=True),
                      jnp.logical_and(lo_ok, has_up))
                pl.when(hi_ok)(
                    lambda z=z_hi, q=q: foreign_z(z, q, up=False).wait_recv()
                )
                start(foreign_z(z_hi, q, up=False),
                      jnp.logical_and(hi_ok, has_dn))

        def opposite_gemms(z_src, cond):
            if cond is None:
                gemm_chunk(j_opp, z_src)
            else:
                pl.when(cond)(lambda: gemm_chunk(j_opp, z_src))

        plane_pieces(opposite_gemms)

        for desc, cond in started:
            if cond is None:
                desc.wait_send()
            else:
                pl.when(cond)(lambda d=desc: d.wait_send())

    dma = pltpu.SemaphoreType.DMA
    return pl.pallas_call(
        body,
        out_shape=jax.ShapeDtypeStruct((m_glob, n_per), jnp.float32),
        in_specs=[
            pl.BlockSpec(memory_space=pltpu.VMEM),
            pl.BlockSpec(memory_space=pltpu.VMEM),
        ],
        out_specs=pl.BlockSpec(memory_space=pltpu.VMEM),
        scratch_shapes=[
            pltpu.VMEM((NF, NZ, NQ, m_q, k), jnp.float32),
            dma,
            dma((NZ, NQ)), dma((NZ, NQ)), dma((NZ, NQ)), dma((NZ, NQ)),
            dma((NZ, NQ)), dma((NZ, NQ)), dma((NZ, NQ)), dma((NZ, NQ)),
            dma((NZ, NQ)), dma((NZ, NQ)), dma((NZ, NQ)), dma((NZ, NQ)),
            dma((NZ, NQ)), dma((NZ, NQ)), dma((NZ, NQ)), dma((NZ, NQ)),
        ],
        compiler_params=pltpu.CompilerParams(collective_id=0),
    )(x, w_mat)


# device time: 77995 ns/iter; 1.0867x vs baseline; 1.0867x over previous
import jax
import jax.numpy as jnp
from jax import lax
from jax.experimental import pallas as pl
from jax.experimental.pallas import tpu as pltpu

N_DEV = 16
NZ = 4
NF = 4
NQ = 4


def kernel(x, w_mat):
    m_per, k = x.shape
    _, n_per = w_mat.shape
    m_glob = N_DEV * m_per
    m_q = m_per // NQ

    def body(x_ref, w_ref, out_ref, gather_ref, copy_sem,
             su, ru, sd, rd,
             sA, rA, sB, rB,
             sC, rC, sD, rD):
        my = lax.axis_index("i")
        jj = lax.rem(my, NF)
        zz = my // NF
        j_left = lax.rem(jj + (NF - 1), NF)
        j_right = lax.rem(jj + 1, NF)
        p_right = NF * zz + j_right
        p_left = NF * zz + j_left
        p_up = my + NF
        p_dn = my - NF

        has_up = zz < NZ - 1
        has_dn = zz > 0

        started = []

        def start(desc, cond=None):
            if cond is None:
                desc.start()
            else:
                pl.when(cond)(lambda: desc.start())
            started.append((desc, cond))

        barrier_sem = pltpu.get_barrier_semaphore()
        for nbr in (p_left, p_right):
            pl.semaphore_signal(barrier_sem, inc=1, device_id=(nbr,),
                                device_id_type=pl.DeviceIdType.MESH)
        pl.when(has_up)(lambda: pl.semaphore_signal(
            barrier_sem, inc=1, device_id=(p_up,),
            device_id_type=pl.DeviceIdType.MESH))
        pl.when(has_dn)(lambda: pl.semaphore_signal(
            barrier_sem, inc=1, device_id=(p_dn,),
            device_id_type=pl.DeviceIdType.MESH))
        pl.semaphore_wait(barrier_sem, 2)
        pl.when(has_up)(lambda: pl.semaphore_wait(barrier_sem, 1))
        pl.when(has_dn)(lambda: pl.semaphore_wait(barrier_sem, 1))

        copies = [
            pltpu.make_async_copy(
                x_ref.at[pl.ds(q * m_q, m_q)],
                gather_ref.at[jj, zz, q],
                copy_sem,
            )
            for q in range(NQ)
        ]
        for c in copies:
            c.start()
        for c in copies:
            c.wait()

        def z_send(z_src, q, up):
            return pltpu.make_async_remote_copy(
                src_ref=gather_ref.at[jj, z_src, q],
                dst_ref=gather_ref.at[jj, z_src, q],
                send_sem=(su if up else sd).at[z_src, q],
                recv_sem=(ru if up else rd).at[z_src, q],
                device_id=(p_up if up else p_dn,),
                device_id_type=pl.DeviceIdType.MESH,
            )

        def face_send(j_src, z_src, q, to_right, sems):
            s_sem, r_sem = sems
            return pltpu.make_async_remote_copy(
                src_ref=gather_ref.at[j_src, z_src, q],
                dst_ref=gather_ref.at[j_src, z_src, q],
                send_sem=s_sem.at[z_src, q],
                recv_sem=r_sem.at[z_src, q],
                device_id=(p_right if to_right else p_left,),
                device_id_type=pl.DeviceIdType.MESH,
            )

        def gemm_rows(piece_rows, origin):
            out_ref[pl.ds(origin * m_per, m_per), :] = jnp.maximum(
                jnp.dot(piece_rows, w_ref[...],
                        preferred_element_type=jnp.float32),
                0.0,
            )

        def gemm_chunk(j_src, z_src):
            gemm_rows(
                gather_ref[j_src, z_src].reshape(m_per, k),
                NF * z_src + j_src,
            )

        for q in range(NQ):
            start(z_send(zz, q, up=True), has_up)
            start(z_send(zz, q, up=False), has_dn)
            start(face_send(jj, zz, q, True, (sA, rA)))
            start(face_send(jj, zz, q, False, (sB, rB)))

        gemm_rows(x_ref[...], my)

        for d in range(1, NZ):
            z_lo = zz - d
            z_hi = zz + d
            lo_ok = z_lo >= 0
            hi_ok = z_hi <= NZ - 1
            for q in range(NQ):
                def lo_work(q=q):
                    pltpu.make_async_remote_copy(
                        src_ref=gather_ref.at[jj, z_lo, q],
                        dst_ref=gather_ref.at[jj, z_lo, q],
                        send_sem=su.at[z_lo, q],
                        recv_sem=ru.at[z_lo, q],
                        device_id=(p_dn,),
                        device_id_type=pl.DeviceIdType.MESH,
                    ).wait_recv()
                pl.when(lo_ok)(lo_work)
                start(z_send(z_lo, q, up=True),
                      jnp.logical_and(lo_ok, has_up))
                start(face_send(jj, z_lo, q, True, (sA, rA)), lo_ok)
                start(face_send(jj, z_lo, q, False, (sB, rB)), lo_ok)

                def hi_work(q=q):
                    pltpu.make_async_remote_copy(
                        src_ref=gather_ref.at[jj, z_hi, q],
                        dst_ref=gather_ref.at[jj, z_hi, q],
                        send_sem=sd.at[z_hi, q],
                        recv_sem=rd.at[z_hi, q],
                        device_id=(p_up,),
                        device_id_type=pl.DeviceIdType.MESH,
                    ).wait_recv()
                pl.when(hi_ok)(hi_work)
                start(z_send(z_hi, q, up=False),
                      jnp.logical_and(hi_ok, has_dn))
                start(face_send(jj, z_hi, q, True, (sA, rA)), hi_ok)
                start(face_send(jj, z_hi, q, False, (sB, rB)), hi_ok)
            pl.when(lo_ok)(lambda z=z_lo: gemm_chunk(jj, z))
            pl.when(hi_ok)(lambda z=z_hi: gemm_chunk(jj, z))

        z_order = [(zz, None)] + [
            (zz - d, zz + d) for d in range(1, NZ)
        ]

        def plane_pieces(process):
            for z_src, z_alt in z_order:
                if z_alt is None:
                    process(z_src, None)
                else:
                    process(z_src, z_src >= 0)
                    process(z_alt, z_alt <= NZ - 1)

        def neighbor_stacks(z_src, cond):
            for q in range(NQ):
                def workA(q=q):
                    face_send(j_left, z_src, q, True, (sA, rA)).wait_recv()
                if cond is None:
                    workA()
                else:
                    pl.when(cond)(workA)
                if q < NQ // 2:
                    start(face_send(j_left, z_src, q, True, (sC, rC)),
                          cond)

                def workB(q=q):
                    face_send(j_right, z_src, q, False, (sB, rB)).wait_recv()
                if cond is None:
                    workB()
                else:
                    pl.when(cond)(workB)
                if q >= NQ // 2:
                    start(face_send(j_right, z_src, q, False, (sD, rD)),
                          cond)

            def gemms():
                gemm_chunk(j_left, z_src)
                gemm_chunk(j_right, z_src)
            if cond is None:
                gemms()
            else:
                pl.when(cond)(gemms)

        plane_pieces(neighbor_stacks)

        j_opp = lax.rem(jj + 2, NF)

        def opposite_halves(z_src, cond):
            for q in range(NQ):
                def work(q=q):
                    if q < NQ // 2:
                        face_send(j_opp, z_src, q, True, (sC, rC)).wait_recv()
                    else:
                        face_send(j_opp, z_src, q, False, (sD, rD)).wait_recv()
                if cond is None:
                    work()
                else:
                    pl.when(cond)(work)

            def gemms():
                gemm_chunk(j_opp, z_src)
            if cond is None:
                gemms()
            else:
                pl.when(cond)(gemms)

        plane_pieces(opposite_halves)

        for desc, cond in started:
            if cond is None:
                desc.wait_send()
            else:
                pl.when(cond)(lambda d=desc: d.wait_send())

    dma = pltpu.SemaphoreType.DMA
    return pl.pallas_call(
        body,
        out_shape=jax.ShapeDtypeStruct((m_glob, n_per), jnp.float32),
        in_specs=[
            pl.BlockSpec(memory_space=pltpu.VMEM),
            pl.BlockSpec(memory_space=pltpu.VMEM),
        ],
        out_specs=pl.BlockSpec(memory_space=pltpu.VMEM),
        scratch_shapes=[
            pltpu.VMEM((NF, NZ, NQ, m_q, k), jnp.float32),
            dma,
            dma((NZ, NQ)), dma((NZ, NQ)), dma((NZ, NQ)), dma((NZ, NQ)),
            dma((NZ, NQ)), dma((NZ, NQ)), dma((NZ, NQ)), dma((NZ, NQ)),
            dma((NZ, NQ)), dma((NZ, NQ)), dma((NZ, NQ)), dma((NZ, NQ)),
        ],
        compiler_params=pltpu.CompilerParams(collective_id=0),
    )(x, w_mat)
